# baseline (speedup 1.0000x reference)
import jax
import jax.numpy as jnp
from jax.experimental import pallas as pl
from jax.experimental.pallas import tpu as pltpu

_N = 4096
_D = 64
_CH = 512
_NCH = _N // _CH


def _pipe_kernel(at, bt, x, ao, bo, a_v, b_v, x_v, in_sem, out_sem, x_sem):
    def in_cps(i):
        sl = pl.ds(i * _CH, _CH)
        return (
            pltpu.make_async_copy(at.at[:, sl], a_v.at[:, sl], in_sem.at[i, 0]),
            pltpu.make_async_copy(bt.at[:, sl], b_v.at[:, sl], in_sem.at[i, 1]),
        )

    def out_cps(i):
        sl = pl.ds(i * _CH, _CH)
        return (
            pltpu.make_async_copy(a_v.at[:, sl], ao.at[:, sl], out_sem.at[i, 0]),
            pltpu.make_async_copy(b_v.at[:, sl], bo.at[:, sl], out_sem.at[i, 1]),
        )

    for i in range(_NCH):
        for c in in_cps(i):
            c.start()
    for i in range(_NCH):
        sl = pl.ds(i * _CH, _CH)
        for c in in_cps(i):
            c.wait()
        for c in out_cps(i):
            c.start()
        x_v[sl] = jnp.sum(a_v[:, sl] * b_v[:, sl], axis=0)
    xc = pltpu.make_async_copy(x_v, x, x_sem)
    xc.start()
    for i in range(_NCH):
        for c in out_cps(i):
            c.wait()
    xc.wait()


def kernel(gum, gim):
    n, d = gum.shape
    at = gum.T
    bt = gim.T
    x, aot, bot = pl.pallas_call(
        _pipe_kernel,
        in_specs=[
            pl.BlockSpec(memory_space=pl.ANY),
            pl.BlockSpec(memory_space=pl.ANY),
        ],
        out_specs=(
            pl.BlockSpec(memory_space=pl.ANY),
            pl.BlockSpec(memory_space=pl.ANY),
            pl.BlockSpec(memory_space=pl.ANY),
        ),
        out_shape=(
            jax.ShapeDtypeStruct((n,), jnp.float32),
            jax.ShapeDtypeStruct((d, n), jnp.float32),
            jax.ShapeDtypeStruct((d, n), jnp.float32),
        ),
        scratch_shapes=[
            pltpu.VMEM((_D, _N), jnp.float32),
            pltpu.VMEM((_D, _N), jnp.float32),
            pltpu.VMEM((_N,), jnp.float32),
            pltpu.SemaphoreType.DMA((_NCH, 2)),
            pltpu.SemaphoreType.DMA((_NCH, 2)),
            pltpu.SemaphoreType.DMA,
        ],
    )(at, bt)
    return (x, aot.T, bot.T)


# speedup vs baseline: 1.0027x; 1.0027x over previous
import jax
import jax.numpy as jnp
from jax.experimental import pallas as pl
from jax.experimental.pallas import tpu as pltpu

_N = 4096
_D = 64
_CH = 2048
_NCH = _N // _CH


def _pipe_kernel(at, bt, x, ao, bo, a_v, b_v, x_v, in_sem, out_sem, x_sem):
    def in_cps(i):
        sl = pl.ds(i * _CH, _CH)
        return (
            pltpu.make_async_copy(at.at[:, sl], a_v.at[:, sl], in_sem.at[i, 0]),
            pltpu.make_async_copy(bt.at[:, sl], b_v.at[:, sl], in_sem.at[i, 1]),
        )

    def out_cps(i):
        sl = pl.ds(i * _CH, _CH)
        return (
            pltpu.make_async_copy(a_v.at[:, sl], ao.at[:, sl], out_sem.at[i, 0]),
            pltpu.make_async_copy(b_v.at[:, sl], bo.at[:, sl], out_sem.at[i, 1]),
        )

    for i in range(_NCH):
        for c in in_cps(i):
            c.start()
    for i in range(_NCH):
        sl = pl.ds(i * _CH, _CH)
        for c in in_cps(i):
            c.wait()
        for c in out_cps(i):
            c.start()
        x_v[sl] = jnp.sum(a_v[:, sl] * b_v[:, sl], axis=0)
    xc = pltpu.make_async_copy(x_v, x, x_sem)
    xc.start()
    for i in range(_NCH):
        for c in out_cps(i):
            c.wait()
    xc.wait()


def kernel(gum, gim):
    n, d = gum.shape
    at = gum.T
    bt = gim.T
    x, aot, bot = pl.pallas_call(
        _pipe_kernel,
        in_specs=[
            pl.BlockSpec(memory_space=pl.ANY),
            pl.BlockSpec(memory_space=pl.ANY),
        ],
        out_specs=(
            pl.BlockSpec(memory_space=pl.ANY),
            pl.BlockSpec(memory_space=pl.ANY),
            pl.BlockSpec(memory_space=pl.ANY),
        ),
        out_shape=(
            jax.ShapeDtypeStruct((n,), jnp.float32),
            jax.ShapeDtypeStruct((d, n), jnp.float32),
            jax.ShapeDtypeStruct((d, n), jnp.float32),
        ),
        scratch_shapes=[
            pltpu.VMEM((_D, _N), jnp.float32),
            pltpu.VMEM((_D, _N), jnp.float32),
            pltpu.VMEM((_N,), jnp.float32),
            pltpu.SemaphoreType.DMA((_NCH, 2)),
            pltpu.SemaphoreType.DMA((_NCH, 2)),
            pltpu.SemaphoreType.DMA,
        ],
    )(at, bt)
    return (x, aot.T, bot.T)


# speedup vs baseline: 1.0293x; 1.0265x over previous
import jax
import jax.numpy as jnp
from jax.experimental import pallas as pl
from jax.experimental.pallas import tpu as pltpu

_N = 4096
_D = 64
_CH = 1024
_NCH = _N // _CH


def _pipe_kernel(at, bt, x, ao, bo, a_v, b_v, x_v, in_sem, out_sem, x_sem):
    def in_cps(i):
        sl = pl.ds(i * _CH, _CH)
        return (
            pltpu.make_async_copy(at.at[:, sl], a_v.at[:, sl], in_sem.at[i, 0]),
            pltpu.make_async_copy(bt.at[:, sl], b_v.at[:, sl], in_sem.at[i, 1]),
        )

    def out_cps(i):
        sl = pl.ds(i * _CH, _CH)
        return (
            pltpu.make_async_copy(a_v.at[:, sl], ao.at[:, sl], out_sem.at[i, 0]),
            pltpu.make_async_copy(b_v.at[:, sl], bo.at[:, sl], out_sem.at[i, 1]),
        )

    for i in range(_NCH):
        for c in in_cps(i):
            c.start()
    for i in range(_NCH):
        sl = pl.ds(i * _CH, _CH)
        for c in in_cps(i):
            c.wait()
        for c in out_cps(i):
            c.start()
        x_v[sl] = jnp.sum(a_v[:, sl] * b_v[:, sl], axis=0)
    xc = pltpu.make_async_copy(x_v, x, x_sem)
    xc.start()
    for i in range(_NCH):
        for c in out_cps(i):
            c.wait()
    xc.wait()


def kernel(gum, gim):
    n, d = gum.shape
    at = gum.T
    bt = gim.T
    x, aot, bot = pl.pallas_call(
        _pipe_kernel,
        in_specs=[
            pl.BlockSpec(memory_space=pl.ANY),
            pl.BlockSpec(memory_space=pl.ANY),
        ],
        out_specs=(
            pl.BlockSpec(memory_space=pl.ANY),
            pl.BlockSpec(memory_space=pl.ANY),
            pl.BlockSpec(memory_space=pl.ANY),
        ),
        out_shape=(
            jax.ShapeDtypeStruct((n,), jnp.float32),
            jax.ShapeDtypeStruct((d, n), jnp.float32),
            jax.ShapeDtypeStruct((d, n), jnp.float32),
        ),
        scratch_shapes=[
            pltpu.VMEM((_D, _N), jnp.float32),
            pltpu.VMEM((_D, _N), jnp.float32),
            pltpu.VMEM((_N,), jnp.float32),
            pltpu.SemaphoreType.DMA((_NCH, 2)),
            pltpu.SemaphoreType.DMA((_NCH, 2)),
            pltpu.SemaphoreType.DMA,
        ],
    )(at, bt)
    return (x, aot.T, bot.T)


# speedup vs baseline: 1.0521x; 1.0222x over previous
import jax
import jax.numpy as jnp
from jax.experimental import pallas as pl
from jax.experimental.pallas import tpu as pltpu

_N = 4096
_D = 64
_KC = 16
_NCH = _D // _KC


def _pipe_kernel(at, bt, x, ao, bo, a_v, b_v, x_v, in_sem, out_sem, x_sem):
    def in_cps(i):
        sl = pl.ds(i * _KC, _KC)
        return (
            pltpu.make_async_copy(at.at[sl, :], a_v.at[sl, :], in_sem.at[i, 0]),
            pltpu.make_async_copy(bt.at[sl, :], b_v.at[sl, :], in_sem.at[i, 1]),
        )

    def out_cps(i):
        sl = pl.ds(i * _KC, _KC)
        return (
            pltpu.make_async_copy(a_v.at[sl, :], ao.at[sl, :], out_sem.at[i, 0]),
            pltpu.make_async_copy(b_v.at[sl, :], bo.at[sl, :], out_sem.at[i, 1]),
        )

    for i in range(_NCH):
        for c in in_cps(i):
            c.start()
    for i in range(_NCH):
        sl = pl.ds(i * _KC, _KC)
        for c in in_cps(i):
            c.wait()
        for c in out_cps(i):
            c.start()
        part = jnp.sum(a_v[sl, :] * b_v[sl, :], axis=0)
        if i == 0:
            x_v[...] = part
        else:
            x_v[...] = x_v[...] + part
    xc = pltpu.make_async_copy(x_v, x, x_sem)
    xc.start()
    for i in range(_NCH):
        for c in out_cps(i):
            c.wait()
    xc.wait()


def kernel(gum, gim):
    n, d = gum.shape
    at = gum.T
    bt = gim.T
    x, aot, bot = pl.pallas_call(
        _pipe_kernel,
        in_specs=[
            pl.BlockSpec(memory_space=pl.ANY),
            pl.BlockSpec(memory_space=pl.ANY),
        ],
        out_specs=(
            pl.BlockSpec(memory_space=pl.ANY),
            pl.BlockSpec(memory_space=pl.ANY),
            pl.BlockSpec(memory_space=pl.ANY),
        ),
        out_shape=(
            jax.ShapeDtypeStruct((n,), jnp.float32),
            jax.ShapeDtypeStruct((d, n), jnp.float32),
            jax.ShapeDtypeStruct((d, n), jnp.float32),
        ),
        scratch_shapes=[
            pltpu.VMEM((_D, _N), jnp.float32),
            pltpu.VMEM((_D, _N), jnp.float32),
            pltpu.VMEM((_N,), jnp.float32),
            pltpu.SemaphoreType.DMA((_NCH, 2)),
            pltpu.SemaphoreType.DMA((_NCH, 2)),
            pltpu.SemaphoreType.DMA,
        ],
    )(at, bt)
    return (x, aot.T, bot.T)
